# baseline (speedup 1.0000x reference)
import functools

import jax
import jax.numpy as jnp
from jax import lax
from jax.experimental import pallas as pl
from jax.experimental.pallas import tpu as pltpu
from jax.experimental.pallas import tpu_sc as plsc

N = 10000
E = 320000
D_IN = 128
HID = 64
NUM_CLASSES = 40

NP = 10240
NCORE = 2
NSUB = 16
NW = NCORE * NSUB
ROWS_PT = NP // NSUB

CHUNK = 48
NBUF = 2
NOUT = 2
E_TOT = E + N
E_PAD = 331776


def _sc_gat_edges(heads):
    head_split = heads == 2
    NIT = E_PAD // ((NSUB if head_split else NW) * CHUNK)
    EPT = NIT * CHUNK
    rv_off = 0 if head_split else 64

    mesh = plsc.VectorSubcoreMesh(core_axis_name="c", subcore_axis_name="s")

    @functools.partial(
        pl.kernel,
        mesh=mesh,
        out_type=jax.ShapeDtypeStruct((NCORE * NP, 128), jnp.float32),
        scratch_types=(
            [pltpu.VMEM((CHUNK,), jnp.int32)] * (2 * NBUF + NOUT)
            + [pltpu.VMEM((CHUNK, 128), jnp.float32)] * (2 * NBUF + NOUT)
            + [
                pltpu.VMEM((128,), jnp.float32),
                pltpu.VMEM_SHARED((NP, 128), jnp.float32),
            ]
            + [pltpu.SemaphoreType.DMA] * (2 * NBUF + NOUT)
        ),
    )
    def k(xl_hbm, xr_hbm, src_hbm, dst_hbm, att_hbm, zn_hbm, num_hbm, *refs):
        srcvs, dstvs = list(refs[0:NBUF]), list(refs[NBUF:2 * NBUF])
        dstS = list(refs[2 * NBUF:2 * NBUF + NOUT])
        o = 2 * NBUF + NOUT
        xsvs = list(refs[o:o + NBUF])
        xrvs = list(refs[o + NBUF:o + 2 * NBUF])
        outvs = list(refs[o + 2 * NBUF:o + 2 * NBUF + NOUT])
        attv, acc = refs[o + 2 * NBUF + NOUT], refs[o + 2 * NBUF + NOUT + 1]
        s = o + 2 * NBUF + NOUT + 2
        semsA = list(refs[s:s + NBUF])
        semsB = list(refs[s + NBUF:s + 2 * NBUF])
        semsS = list(refs[s + 2 * NBUF:s + 2 * NBUF + NOUT])
        cid = lax.axis_index("c")
        sid = lax.axis_index("s")

        lane = lax.iota(jnp.int32, 16)
        z16 = jnp.zeros((16,), jnp.float32)

        r0 = sid * ROWS_PT
        pltpu.sync_copy(zn_hbm.at[pl.ds(r0, ROWS_PT)], acc.at[pl.ds(r0, ROWS_PT)])
        pltpu.sync_copy(att_hbm, attv)
        plsc.subcore_barrier()

        s_off = cid * 64 if head_split else 0
        att_regs = [attv[pl.ds(s_off + 16 * j, 16)] for j in range(4)]

        perm_idx = [jnp.bitwise_xor(lane, sh) for sh in (8, 4, 2, 1)]
        dn = lax.GatherDimensionNumbers(
            offset_dims=(), collapsed_slice_dims=(0,), start_index_map=(0,))

        def _bcast(v, idx):
            return lax.gather(v, idx[:, None], dn, (1,),
                              mode=lax.GatherScatterMode.PROMISE_IN_BOUNDS)

        def _hsum(v):
            for idx in perm_idx:
                v = v + _bcast(v, idx)
            return v

        ebase = (sid if head_split else sid * NCORE + cid) * EPT

        def issue(iv, b):
            base = ebase + iv * CHUNK
            pltpu.sync_copy(src_hbm.at[pl.ds(base, CHUNK)], srcvs[b])
            pltpu.sync_copy(dst_hbm.at[pl.ds(base, CHUNK)], dstvs[b])
            pltpu.async_copy(xl_hbm.at[srcvs[b]], xsvs[b], semsA[b])
            pltpu.async_copy(xr_hbm.at[dstvs[b]], xrvs[b], semsB[b])

        def make_e_body(xsv, xrv, outv):
            def e_body(e, c2):
                svs = [xsv[e, pl.ds(s_off + 16 * j, 16)] for j in range(4)]
                rvs = [xrv[e, pl.ds(s_off + rv_off + 16 * j, 16)] for j in range(4)]
                logit = z16
                for j in range(4):
                    a = svs[j] + rvs[j]
                    a = jnp.maximum(a, 0.2 * a)
                    logit = logit + a * att_regs[j]
                ex = jnp.exp(_hsum(logit))
                for j in range(4):
                    outv[e, pl.ds(16 * j, 16)] = svs[j] * ex
                outv[e, pl.ds(64, 16)] = jnp.where(lane == 0, ex, z16)
                return c2

            return e_body

        issue(0, 0)

        def g_body(g, carry):
            for b in range(NBUF):
                it = g * NBUF + b
                pltpu.make_async_copy(xl_hbm.at[srcvs[b]], xsvs[b], semsA[b]).wait()
                pltpu.make_async_copy(xr_hbm.at[dstvs[b]], xrvs[b], semsB[b]).wait()

                @pl.when(g >= 1)
                def _():
                    pltpu.make_async_copy(outvs[b], acc.at[dstS[b]], semsS[b]).wait()

                for jj in range(CHUNK // 16):
                    dstS[b][pl.ds(16 * jj, 16)] = dstvs[b][pl.ds(16 * jj, 16)]

                @pl.when(it + 1 < NIT)
                def _():
                    issue(it + 1, 1 - b)

                e_body = make_e_body(xsvs[b], xrvs[b], outvs[b])

                @plsc.parallel_loop(0, CHUNK, 1, unroll=4)
                def _(e):
                    e_body(e, 0)
                pltpu.async_copy(outvs[b], acc.at[dstS[b]], semsS[b], add=True)
            return carry

        lax.fori_loop(0, NIT // NBUF, g_body, 0)
        for b in range(NOUT):
            pltpu.make_async_copy(outvs[b], acc.at[dstS[b]], semsS[b]).wait()
        plsc.subcore_barrier()

        pltpu.sync_copy(acc.at[pl.ds(r0, ROWS_PT)],
                        num_hbm.at[pl.ds(cid * NP + r0, ROWS_PT)])

    return k


def _mm2(xp, Wl, bl, Wr, br, bm=512):
    m, kdim = xp.shape
    dout = Wl.shape[1]

    def body(x_ref, wl_ref, bl_ref, wr_ref, br_ref, ol_ref, or_ref):
        xb = x_ref[...]
        ol_ref[...] = jnp.dot(xb, wl_ref[...], preferred_element_type=jnp.float32) + bl_ref[...]
        or_ref[...] = jnp.dot(xb, wr_ref[...], preferred_element_type=jnp.float32) + br_ref[...]

    return pl.pallas_call(
        body,
        grid=(m // bm,),
        in_specs=[
            pl.BlockSpec((bm, kdim), lambda i: (i, 0)),
            pl.BlockSpec((kdim, dout), lambda i: (0, 0)),
            pl.BlockSpec((1, dout), lambda i: (0, 0)),
            pl.BlockSpec((kdim, dout), lambda i: (0, 0)),
            pl.BlockSpec((1, dout), lambda i: (0, 0)),
        ],
        out_specs=[
            pl.BlockSpec((bm, dout), lambda i: (i, 0)),
            pl.BlockSpec((bm, dout), lambda i: (i, 0)),
        ],
        out_shape=[jax.ShapeDtypeStruct((m, dout), jnp.float32)] * 2,
    )(xp, Wl, bl.reshape(1, -1), Wr, br.reshape(1, -1))


def _combine(num_p, heads, bias_in, W, b, bm=512):
    dout = W.shape[1]

    def body(np_ref, b_ref, w_ref, bb_ref, o_ref):
        if heads == 2:
            h0 = np_ref[0, :, 0:64] / (np_ref[0, :, 64:65] + 1e-16)
            h1 = np_ref[1, :, 0:64] / (np_ref[1, :, 64:65] + 1e-16)
            feat = jnp.concatenate([h0, h1], axis=1)
        else:
            s = np_ref[0] + np_ref[1]
            feat = s[:, 0:64] / (s[:, 64:65] + 1e-16)
        h = jnp.maximum(feat + b_ref[...], 0.0)
        o_ref[...] = jnp.dot(h, w_ref[...], preferred_element_type=jnp.float32) + bb_ref[...]

    return pl.pallas_call(
        body,
        grid=(NP // bm,),
        in_specs=[
            pl.BlockSpec((2, bm, 128), lambda i: (0, i, 0)),
            pl.BlockSpec((1, heads * HID), lambda i: (0, 0)),
            pl.BlockSpec((heads * HID, dout), lambda i: (0, 0)),
            pl.BlockSpec((1, dout), lambda i: (0, 0)),
        ],
        out_specs=pl.BlockSpec((bm, dout), lambda i: (i, 0)),
        out_shape=jax.ShapeDtypeStruct((NP, dout), jnp.float32),
    )(num_p, bias_in.reshape(1, -1), W, b.reshape(1, -1))


def kernel(x, edge_index, Wl1, bl1, Wr1, br1, att1, bias1,
           Wl2, bl2, Wr2, br2, att2, bias2, fcW, fcb):
    xp = jnp.zeros((NP, D_IN), jnp.float32).at[:N].set(x)
    loop = jnp.arange(N, dtype=jnp.int32)
    pad = E_PAD - E_TOT
    padi = jnp.arange(pad, dtype=jnp.int32)
    src = jnp.concatenate([edge_index[0], loop, padi * 97 % N])
    dst = jnp.concatenate([edge_index[1], loop, N + padi % (NP - N)])

    zn = jnp.zeros((NP, 128), jnp.float32)

    xl1, xr1 = _mm2(xp, Wl1, bl1, Wr1, br1)
    num1 = _sc_gat_edges(2)(xl1, xr1, src, dst, att1.reshape(-1), zn)
    num1 = num1.reshape(NCORE, NP, 128)

    W2 = jnp.concatenate([Wl2, Wr2], axis=1)
    b2 = jnp.concatenate([bl2, br2])
    att2p = jnp.concatenate([att2.reshape(-1), jnp.zeros((HID,), jnp.float32)])
    t2 = _combine(num1, 2, bias1, W2, b2)
    num2 = _sc_gat_edges(1)(t2, t2, src, dst, att2p, zn)
    num2 = num2.reshape(NCORE, NP, 128)

    out = _combine(num2, 1, bias2, fcW, fcb)
    return out[:N]


# speedup vs baseline: 1.2340x; 1.2340x over previous
import functools

import jax
import jax.numpy as jnp
from jax import lax
from jax.experimental import pallas as pl
from jax.experimental.pallas import tpu as pltpu
from jax.experimental.pallas import tpu_sc as plsc

N = 10000
E = 320000
D_IN = 128
HID = 64
NUM_CLASSES = 40

NP = 10240
NCORE = 2
NSUB = 16
NW = NCORE * NSUB
ROWS_PT = NP // NSUB

CHUNK = 48
NBUF = 3
E_TOT = E + N
E_PAD = 331776


def _sc_gat_edges(heads):
    head_split = heads == 2
    NIT = E_PAD // ((NSUB if head_split else NW) * CHUNK)
    EPT = NIT * CHUNK
    rv_off = 0 if head_split else 64

    mesh = plsc.VectorSubcoreMesh(core_axis_name="c", subcore_axis_name="s")

    @functools.partial(
        pl.kernel,
        mesh=mesh,
        out_type=jax.ShapeDtypeStruct((NCORE * NP, 128), jnp.float32),
        scratch_types=(
            [pltpu.VMEM((CHUNK,), jnp.int32)] * (2 * NBUF)
            + [pltpu.VMEM((CHUNK, 128), jnp.float32)] * (2 * NBUF)
            + [
                pltpu.VMEM((128,), jnp.float32),
                pltpu.VMEM_SHARED((NP, 128), jnp.float32),
            ]
            + [pltpu.SemaphoreType.DMA] * (2 * NBUF)
        ),
    )
    def k(xl_hbm, xr_hbm, src_hbm, dst_hbm, att_hbm, zn_hbm, num_hbm, *refs):
        srcvs, dstvs = list(refs[0:NBUF]), list(refs[NBUF:2 * NBUF])
        xsvs = list(refs[2 * NBUF:3 * NBUF])
        xrvs = list(refs[3 * NBUF:4 * NBUF])
        attv, acc = refs[4 * NBUF], refs[4 * NBUF + 1]
        semsA = list(refs[4 * NBUF + 2:5 * NBUF + 2])
        semsB = list(refs[5 * NBUF + 2:6 * NBUF + 2])
        cid = lax.axis_index("c")
        sid = lax.axis_index("s")

        lane = lax.iota(jnp.int32, 16)
        z16 = jnp.zeros((16,), jnp.float32)

        r0 = sid * ROWS_PT
        pltpu.sync_copy(zn_hbm.at[pl.ds(r0, ROWS_PT)], acc.at[pl.ds(r0, ROWS_PT)])
        pltpu.sync_copy(att_hbm, attv)
        plsc.subcore_barrier()

        s_off = cid * 64 if head_split else 0
        att_regs = [attv[pl.ds(s_off + 16 * j, 16)] for j in range(4)]

        perm_idx = [jnp.bitwise_xor(lane, sh) for sh in (8, 4, 2, 1)]
        dn = lax.GatherDimensionNumbers(
            offset_dims=(), collapsed_slice_dims=(0,), start_index_map=(0,))

        def _bcast(v, idx):
            return lax.gather(v, idx[:, None], dn, (1,),
                              mode=lax.GatherScatterMode.PROMISE_IN_BOUNDS)

        def _hsum(v):
            for idx in perm_idx:
                v = v + _bcast(v, idx)
            return v

        ebase = (sid if head_split else sid * NCORE + cid) * EPT

        def issue(iv, b):
            base = ebase + iv * CHUNK
            pltpu.sync_copy(src_hbm.at[pl.ds(base, CHUNK)], srcvs[b])
            pltpu.sync_copy(dst_hbm.at[pl.ds(base, CHUNK)], dstvs[b])
            pltpu.async_copy(xl_hbm.at[srcvs[b]], xsvs[b], semsA[b])
            pltpu.async_copy(xr_hbm.at[dstvs[b]], xrvs[b], semsB[b])

        def make_e_body(xsv, xrv):
            outv = xrv
            def e_body(e, c2):
                svs = [xsv[e, pl.ds(s_off + 16 * j, 16)] for j in range(4)]
                rvs = [xrv[e, pl.ds(s_off + rv_off + 16 * j, 16)] for j in range(4)]
                logit = z16
                for j in range(4):
                    a = svs[j] + rvs[j]
                    a = jnp.maximum(a, 0.2 * a)
                    logit = logit + a * att_regs[j]
                ex = jnp.exp(_hsum(logit))
                for j in range(4):
                    outv[e, pl.ds(16 * j, 16)] = svs[j] * ex
                outv[e, pl.ds(64, 16)] = jnp.where(lane == 0, ex, z16)
                return c2

            return e_body

        for p in range(NBUF - 1):
            issue(p, p)

        def g_body(g, carry):
            for b in range(NBUF):
                it = g * NBUF + b
                pltpu.make_async_copy(xl_hbm.at[srcvs[b]], xsvs[b], semsA[b]).wait()
                pltpu.make_async_copy(xr_hbm.at[dstvs[b]], xrvs[b], semsB[b]).wait()

                @pl.when(it + NBUF - 1 < NIT)
                def _():
                    issue(it + NBUF - 1, (b + NBUF - 1) % NBUF)

                e_body = make_e_body(xsvs[b], xrvs[b])

                @plsc.parallel_loop(0, CHUNK, 1, unroll=4)
                def _(e):
                    e_body(e, 0)
                pltpu.sync_copy(xrvs[b], acc.at[dstvs[b]], add=True)
            return carry

        lax.fori_loop(0, NIT // NBUF, g_body, 0)
        plsc.subcore_barrier()

        pltpu.sync_copy(acc.at[pl.ds(r0, ROWS_PT)],
                        num_hbm.at[pl.ds(cid * NP + r0, ROWS_PT)])

    return k


def _mm2(xp, Wl, bl, Wr, br, bm=512):
    m, kdim = xp.shape
    dout = Wl.shape[1]

    def body(x_ref, wl_ref, bl_ref, wr_ref, br_ref, ol_ref, or_ref):
        xb = x_ref[...]
        ol_ref[...] = jnp.dot(xb, wl_ref[...], preferred_element_type=jnp.float32) + bl_ref[...]
        or_ref[...] = jnp.dot(xb, wr_ref[...], preferred_element_type=jnp.float32) + br_ref[...]

    return pl.pallas_call(
        body,
        grid=(m // bm,),
        in_specs=[
            pl.BlockSpec((bm, kdim), lambda i: (i, 0)),
            pl.BlockSpec((kdim, dout), lambda i: (0, 0)),
            pl.BlockSpec((1, dout), lambda i: (0, 0)),
            pl.BlockSpec((kdim, dout), lambda i: (0, 0)),
            pl.BlockSpec((1, dout), lambda i: (0, 0)),
        ],
        out_specs=[
            pl.BlockSpec((bm, dout), lambda i: (i, 0)),
            pl.BlockSpec((bm, dout), lambda i: (i, 0)),
        ],
        out_shape=[jax.ShapeDtypeStruct((m, dout), jnp.float32)] * 2,
    )(xp, Wl, bl.reshape(1, -1), Wr, br.reshape(1, -1))


def _combine(num_p, heads, bias_in, W, b, bm=512):
    dout = W.shape[1]

    def body(np_ref, b_ref, w_ref, bb_ref, o_ref):
        if heads == 2:
            h0 = np_ref[0, :, 0:64] / (np_ref[0, :, 64:65] + 1e-16)
            h1 = np_ref[1, :, 0:64] / (np_ref[1, :, 64:65] + 1e-16)
            feat = jnp.concatenate([h0, h1], axis=1)
        else:
            s = np_ref[0] + np_ref[1]
            feat = s[:, 0:64] / (s[:, 64:65] + 1e-16)
        h = jnp.maximum(feat + b_ref[...], 0.0)
        o_ref[...] = jnp.dot(h, w_ref[...], preferred_element_type=jnp.float32) + bb_ref[...]

    return pl.pallas_call(
        body,
        grid=(NP // bm,),
        in_specs=[
            pl.BlockSpec((2, bm, 128), lambda i: (0, i, 0)),
            pl.BlockSpec((1, heads * HID), lambda i: (0, 0)),
            pl.BlockSpec((heads * HID, dout), lambda i: (0, 0)),
            pl.BlockSpec((1, dout), lambda i: (0, 0)),
        ],
        out_specs=pl.BlockSpec((bm, dout), lambda i: (i, 0)),
        out_shape=jax.ShapeDtypeStruct((NP, dout), jnp.float32),
    )(num_p, bias_in.reshape(1, -1), W, b.reshape(1, -1))


def kernel(x, edge_index, Wl1, bl1, Wr1, br1, att1, bias1,
           Wl2, bl2, Wr2, br2, att2, bias2, fcW, fcb):
    xp = jnp.zeros((NP, D_IN), jnp.float32).at[:N].set(x)
    loop = jnp.arange(N, dtype=jnp.int32)
    pad = E_PAD - E_TOT
    padi = jnp.arange(pad, dtype=jnp.int32)
    src = jnp.concatenate([edge_index[0], loop, padi * 97 % N])
    dst = jnp.concatenate([edge_index[1], loop, N + padi % (NP - N)])

    zn = jnp.zeros((NP, 128), jnp.float32)

    xl1, xr1 = _mm2(xp, Wl1, bl1, Wr1, br1)
    num1 = _sc_gat_edges(2)(xl1, xr1, src, dst, att1.reshape(-1), zn)
    num1 = num1.reshape(NCORE, NP, 128)

    W2 = jnp.concatenate([Wl2, Wr2], axis=1)
    b2 = jnp.concatenate([bl2, br2])
    att2p = jnp.concatenate([att2.reshape(-1), jnp.zeros((HID,), jnp.float32)])
    t2 = _combine(num1, 2, bias1, W2, b2)
    num2 = _sc_gat_edges(1)(t2, t2, src, dst, att2p, zn)
    num2 = num2.reshape(NCORE, NP, 128)

    out = _combine(num2, 1, bias2, fcW, fcb)
    return out[:N]
